# baseline (speedup 1.0000x reference)
import functools

import jax
import jax.numpy as jnp
from jax import lax
from jax.experimental import pallas as pl
from jax.experimental.pallas import tpu as pltpu
from jax.experimental.pallas import tpu_sc as plsc

_N = 10000
_E = 320000
_D = 128
_NC = 2
_NS = 16
_NW = _NC * _NS
_K = 80
_EPW = _E // _NW
_NCH = _EPW // _K
_ZR = 80
_NZC = _N // _ZR
_F32 = jnp.float32


def _pre_body(h_ref, w_ref, al_ref, ar_ref, feat_ref, sctab_ref, cvec_ref):
    h = h_ref[...]
    feat = jnp.dot(h, w_ref[...], preferred_element_type=_F32)
    feat_ref[...] = feat
    el = jnp.dot(feat, al_ref[...], preferred_element_type=_F32)
    er = jnp.dot(feat, ar_ref[...], preferred_element_type=_F32)
    sctab_ref[...] = jnp.concatenate(
        [el, er, jnp.zeros((_N, 6), _F32)], axis=1)
    m = jnp.max(el) + jnp.max(er)
    c = jnp.where(m >= 0.0, m, 0.2 * m)
    cvec_ref[...] = jnp.full((1, 16), c, _F32)


_pre_call = pl.pallas_call(
    _pre_body,
    out_shape=[
        jax.ShapeDtypeStruct((_N, _D), _F32),
        jax.ShapeDtypeStruct((_N, 8), _F32),
        jax.ShapeDtypeStruct((1, 16), _F32),
    ],
)


def _sc_edge_body(feat_hbm, sctab_hbm, cvec_hbm, src_hbm, dst_hbm,
                  numer_hbm, denom_hbm,
                  numer_s, denom_s, sctab_s,
                  rows0, rows1, rows2, ex0, ex1,
                  ids0, ids1, idc0, idc1, idc2,
                  scs0, scs1, scd0, scd1, exs, cv,
                  semis0, semis1, semic0, semic1, semic2,
                  semgs0, semgs1, semgd0, semgd1, semgf0, semgf1,
                  semcn0, semcn1, semcn2, semcd0, semcd1, semcd2):
    rows = [rows0, rows1, rows2]
    ex8 = [ex0, ex1]
    idxs = [ids0, ids1]
    idxc = [idc0, idc1, idc2]
    scs = [scs0, scs1]
    scd = [scd0, scd1]
    semis = [semis0, semis1]
    semic = [semic0, semic1, semic2]
    semgs = [semgs0, semgs1]
    semgd = [semgd0, semgd1]
    semgf = [semgf0, semgf1]
    semcn = [semcn0, semcn1, semcn2]
    semcd = [semcd0, semcd1, semcd2]

    cid = lax.axis_index("c")
    sid = lax.axis_index("s")
    wid = cid * _NS + sid
    zero16 = jnp.zeros((16,), _F32)
    lane = lax.iota(jnp.int32, 16)
    lane_hi = lane // 8
    lane_lo = lane % 8

    def zb_body(i, carry):
        for v in range(_D // 16):
            rows0[i, pl.ds(v * 16, 16)] = zero16
        return carry

    lax.fori_loop(0, _ZR, zb_body, 0)

    def zs_body(i, carry):
        plsc.store_scatter(ex0, [i * 2 + lane_hi, lane_lo], zero16)
        return carry

    lax.fori_loop(0, _K // 2, zs_body, 0)
    for k in range((_NZC + _NS - 1) // _NS):
        ci = sid + _NS * k

        @pl.when(ci < _NZC)
        def _zero_chunk():
            r = pl.multiple_of(ci * _ZR, 8)
            pltpu.sync_copy(rows0, numer_s.at[pl.ds(r, _ZR)])
            pltpu.sync_copy(ex0, denom_s.at[pl.ds(r, _ZR)])
            pltpu.sync_copy(sctab_hbm.at[pl.ds(r, _ZR)], scs0)
            pltpu.sync_copy(scs0, sctab_s.at[pl.ds(r, _ZR)])

    pltpu.sync_copy(cvec_hbm.at[0], cv)
    plsc.subcore_barrier()

    cvv = cv[pl.ds(0, 16)]
    base = wid * _EPW
    zeros_i = jnp.zeros((16,), jnp.int32)
    ones_i = jnp.ones((16,), jnp.int32)

    def chunk_off(c):
        return pl.multiple_of(base + c * _K, 8)

    def issue_idx(c, p2, p3):
        off = chunk_off(c)
        pltpu.async_copy(src_hbm.at[pl.ds(off, _K)], idxs[p2], semis[p2])
        pltpu.async_copy(dst_hbm.at[pl.ds(off, _K)], idxc[p3], semic[p3])

    def wait_idx(c, p2, p3):
        off = chunk_off(c)
        pltpu.make_async_copy(
            src_hbm.at[pl.ds(off, _K)], idxs[p2], semis[p2]).wait()
        pltpu.make_async_copy(
            dst_hbm.at[pl.ds(off, _K)], idxc[p3], semic[p3]).wait()

    def issue_gathers(p2, p3):
        pltpu.async_copy(sctab_s.at[idxs[p2]], scs[p2], semgs[p2])
        pltpu.async_copy(sctab_s.at[idxc[p3]], scd[p2], semgd[p2])
        pltpu.async_copy(feat_hbm.at[idxs[p2]], rows[p3], semgf[p2])

    def issue_scatter(p3, p2):
        pltpu.async_copy(rows[p3], numer_s.at[idxc[p3]], semcn[p3], add=True)
        pltpu.async_copy(ex8[p2], denom_s.at[idxc[p3]], semcd[p3], add=True)

    def wait_scatter(p3, p2):
        pltpu.make_async_copy(
            rows[p3], numer_s.at[idxc[p3]], semcn[p3]).wait()
        pltpu.make_async_copy(
            ex8[p2], denom_s.at[idxc[p3]], semcd[p3]).wait()

    def compute(p2, p3):
        pltpu.make_async_copy(
            sctab_s.at[idxs[p2]], scs[p2], semgs[p2]).wait()
        pltpu.make_async_copy(
            sctab_s.at[idxc[p3]], scd[p2], semgd[p2]).wait()

        def ex_body(j, carry):
            rid = j * 16 + lane
            el = plsc.load_gather(scs[p2], [rid, zeros_i])
            er = plsc.load_gather(scd[p2], [rid, ones_i])
            s = el + er
            e = jnp.where(s >= 0.0, s, 0.2 * s)
            exs[pl.ds(j * 16, 16)] = jnp.exp(e - cvv)
            return carry

        lax.fori_loop(0, _K // 16, ex_body, 0)
        pltpu.make_async_copy(
            feat_hbm.at[idxs[p2]], rows[p3], semgf[p2]).wait()

        def sc_body(j, carry):
            exv = exs[pl.ds(j * 16, 16)]
            for m in range(8):
                e0 = j * 16 + 2 * m
                bv0 = jnp.full((16,), exv[2 * m], _F32)
                bv1 = jnp.full((16,), exv[2 * m + 1], _F32)
                bvp = jnp.where(lane_hi == 0, bv0, bv1)
                plsc.store_scatter(ex8[p2], [e0 + lane_hi, lane_lo], bvp)
                for v in range(_D // 16):
                    rows[p3][e0, pl.ds(v * 16, 16)] = (
                        rows[p3][e0, pl.ds(v * 16, 16)] * bv0)
                    rows[p3][e0 + 1, pl.ds(v * 16, 16)] = (
                        rows[p3][e0 + 1, pl.ds(v * 16, 16)] * bv1)
            return carry

        lax.fori_loop(0, _K // 16, sc_body, 0)

    def body(c, p2, p3, first=False, has_next=True, has_next2=True):
        q2, q3 = (p2 + 1) % 2, (p3 + 1) % 3
        if has_next:
            wait_idx(c + 1, q2, q3)
            issue_gathers(q2, q3)
        compute(p2, p3)
        if not first:
            wait_scatter((p3 + 2) % 3, q2)
        issue_scatter(p3, p2)
        if has_next2:
            issue_idx(c + 2, p2, (p3 + 2) % 3)

    issue_idx(0, 0, 0)
    issue_idx(1, 1, 1)
    wait_idx(0, 0, 0)
    issue_gathers(0, 0)
    body(0, 0, 0, first=True)

    def step(i, carry):
        c = 1 + 6 * i
        for j in range(6):
            body(c + j, (1 + j) % 2, (1 + j) % 3)
        return carry

    lax.fori_loop(0, (_NCH - 5) // 6, step, 0)

    body(121, 1, 1)
    body(122, 0, 2)
    body(123, 1, 0, has_next2=False)
    body(124, 0, 1, has_next=False, has_next2=False)
    wait_scatter(1, 0)
    plsc.subcore_barrier()

    for k in range((_NZC + _NS - 1) // _NS):
        ci = sid + _NS * k

        @pl.when(ci < _NZC)
        def _copy_chunk():
            r = pl.multiple_of(ci * _ZR, 8)
            pltpu.sync_copy(numer_s.at[pl.ds(r, _ZR)], rows0)
            pltpu.sync_copy(rows0, numer_hbm.at[cid].at[pl.ds(r, _ZR)])
            pltpu.sync_copy(denom_s.at[pl.ds(r, _ZR)], ex0)
            pltpu.sync_copy(ex0, denom_hbm.at[cid].at[pl.ds(r, _ZR)])


_edge_call = pl.kernel(
    _sc_edge_body,
    out_type=[
        jax.ShapeDtypeStruct((_NC, _N, _D), _F32),
        jax.ShapeDtypeStruct((_NC, _N, 8), _F32),
    ],
    mesh=plsc.VectorSubcoreMesh(core_axis_name="c", subcore_axis_name="s"),
    compiler_params=pltpu.CompilerParams(needs_layout_passes=False,
                                         use_tc_tiling_on_sc=False),
    scratch_types=[
        pltpu.VMEM_SHARED((_N, _D), _F32),
        pltpu.VMEM_SHARED((_N, 8), _F32),
        pltpu.VMEM_SHARED((_N, 8), _F32),
        pltpu.VMEM((_K, _D), _F32),
        pltpu.VMEM((_K, _D), _F32),
        pltpu.VMEM((_K, _D), _F32),
        pltpu.VMEM((_K, 8), _F32),
        pltpu.VMEM((_K, 8), _F32),
        pltpu.VMEM((_K,), jnp.int32),
        pltpu.VMEM((_K,), jnp.int32),
        pltpu.VMEM((_K,), jnp.int32),
        pltpu.VMEM((_K,), jnp.int32),
        pltpu.VMEM((_K,), jnp.int32),
        pltpu.VMEM((_K, 8), _F32),
        pltpu.VMEM((_K, 8), _F32),
        pltpu.VMEM((_K, 8), _F32),
        pltpu.VMEM((_K, 8), _F32),
        pltpu.VMEM((_K,), _F32),
        pltpu.VMEM((16,), _F32),
        pltpu.SemaphoreType.DMA,
        pltpu.SemaphoreType.DMA,
        pltpu.SemaphoreType.DMA,
        pltpu.SemaphoreType.DMA,
        pltpu.SemaphoreType.DMA,
        pltpu.SemaphoreType.DMA,
        pltpu.SemaphoreType.DMA,
        pltpu.SemaphoreType.DMA,
        pltpu.SemaphoreType.DMA,
        pltpu.SemaphoreType.DMA,
        pltpu.SemaphoreType.DMA,
        pltpu.SemaphoreType.DMA,
        pltpu.SemaphoreType.DMA,
        pltpu.SemaphoreType.DMA,
        pltpu.SemaphoreType.DMA,
        pltpu.SemaphoreType.DMA,
        pltpu.SemaphoreType.DMA,
    ],
)


def _post_body(numer_ref, denom_ref, h_ref, b_ref, g_ref, be_ref, out_ref,
               *, act):
    numer = numer_ref[0] + numer_ref[1]
    den = denom_ref[0, :, 0:1] + denom_ref[1, :, 0:1]
    v = numer / (den + 1e-30) + h_ref[...] + b_ref[...]
    if act:
        v = jnp.maximum(v, 0.0)
    mu = jnp.mean(v, axis=0, keepdims=True)
    var = jnp.mean((v - mu) ** 2, axis=0, keepdims=True)
    out_ref[...] = (v - mu) * lax.rsqrt(var + 1e-5) * g_ref[...] + be_ref[...]


def _make_post(act):
    return pl.pallas_call(
        functools.partial(_post_body, act=act),
        out_shape=jax.ShapeDtypeStruct((_N, _D), _F32),
    )


_post_act = _make_post(True)
_post_noact = _make_post(False)


def kernel(x, edge_index, W0, al0, ar0, b0, g0, be0,
           W1, al1, ar1, b1, g1, be1, W2, al2, ar2, b2, g2, be2):
    src = edge_index[0].astype(jnp.int32)
    dst = edge_index[1].astype(jnp.int32)
    h = x
    layers = [
        (W0, al0, ar0, b0, g0, be0, True),
        (W1, al1, ar1, b1, g1, be1, True),
        (W2, al2, ar2, b2, g2, be2, False),
    ]
    for W, al, ar, b, g, be, act in layers:
        feat, sctab, cvec = _pre_call(h, W, al.reshape(_D, 1),
                                      ar.reshape(_D, 1))
        numer, denom = _edge_call(feat, sctab, cvec, src, dst)
        post = _post_act if act else _post_noact
        h = post(numer, denom, h, b.reshape(1, _D), g.reshape(1, _D),
                 be.reshape(1, _D))
    return h


# speedup vs baseline: 1.5552x; 1.5552x over previous
import functools

import jax
import jax.numpy as jnp
from jax import lax
from jax.experimental import pallas as pl
from jax.experimental.pallas import tpu as pltpu
from jax.experimental.pallas import tpu_sc as plsc

_N = 10000
_E = 320000
_D = 128
_NC = 2
_NS = 16
_NW = _NC * _NS
_K = 80
_EPW = _E // _NW
_NCH = _EPW // _K
_ZR = 80
_NZC = _N // _ZR
_F32 = jnp.float32


def _pre_body(h_ref, w_ref, al_ref, ar_ref, feat_ref, sctab_ref, cvec_ref):
    h = h_ref[...]
    feat = jnp.dot(h, w_ref[...], preferred_element_type=_F32)
    feat_ref[...] = feat
    el = jnp.dot(feat, al_ref[...], preferred_element_type=_F32)
    er = jnp.dot(feat, ar_ref[...], preferred_element_type=_F32)
    sctab_ref[...] = jnp.concatenate(
        [el, er, jnp.zeros((_N, 6), _F32)], axis=1)
    m = jnp.max(el) + jnp.max(er)
    c = jnp.where(m >= 0.0, m, 0.2 * m)
    cvec_ref[...] = jnp.full((1, 16), c, _F32)


_pre_call = pl.pallas_call(
    _pre_body,
    out_shape=[
        jax.ShapeDtypeStruct((_N, _D), _F32),
        jax.ShapeDtypeStruct((_N, 8), _F32),
        jax.ShapeDtypeStruct((1, 16), _F32),
    ],
)


def _sc_edge_body(feat_hbm, sctab_hbm, cvec_hbm, src_hbm, dst_hbm,
                  numer_hbm, denom_hbm,
                  numer_s, denom_s, sctab_s,
                  rows0, rows1, rows2, ex0,
                  ids0, ids1, idc0, idc1, idc2,
                  scs0, scs1, scd0, scd1, exs, cv,
                  semis0, semis1, semic0, semic1, semic2,
                  semgs0, semgs1, semgd0, semgd1, semgf0, semgf1):
    rows = [rows0, rows1, rows2]
    idxs = [ids0, ids1]
    idxc = [idc0, idc1, idc2]
    scs = [scs0, scs1]
    scd = [scd0, scd1]
    semis = [semis0, semis1]
    semic = [semic0, semic1, semic2]
    semgs = [semgs0, semgs1]
    semgd = [semgd0, semgd1]
    semgf = [semgf0, semgf1]

    cid = lax.axis_index("c")
    sid = lax.axis_index("s")
    wid = cid * _NS + sid
    zero16 = jnp.zeros((16,), _F32)
    lane = lax.iota(jnp.int32, 16)
    lane_hi = lane // 8
    lane_lo = lane % 8

    def zb_body(i, carry):
        for v in range(_D // 16):
            rows0[i, pl.ds(v * 16, 16)] = zero16
        return carry

    lax.fori_loop(0, _ZR, zb_body, 0)

    def zs_body(i, carry):
        ex0[i, pl.ds(0, 16)] = zero16
        return carry

    lax.fori_loop(0, _K, zs_body, 0)
    for k in range((_NZC + _NS - 1) // _NS):
        ci = sid + _NS * k

        @pl.when(ci < _NZC)
        def _zero_chunk():
            r = pl.multiple_of(ci * _ZR, 8)
            pltpu.sync_copy(rows0, numer_s.at[pl.ds(r, _ZR)])
            pltpu.sync_copy(ex0, denom_s.at[pl.ds(r, _ZR)])
            pltpu.sync_copy(sctab_hbm.at[pl.ds(r, _ZR)], scs0)
            pltpu.sync_copy(scs0, sctab_s.at[pl.ds(r, _ZR)])

    pltpu.sync_copy(cvec_hbm.at[0], cv)
    plsc.subcore_barrier()

    cvv = cv[pl.ds(0, 16)]
    base = wid * _EPW
    zeros_i = jnp.zeros((16,), jnp.int32)
    ones_i = jnp.ones((16,), jnp.int32)

    def chunk_off(c):
        return pl.multiple_of(base + c * _K, 8)

    def issue_idx(c, p2, p3):
        off = chunk_off(c)
        pltpu.async_copy(src_hbm.at[pl.ds(off, _K)], idxs[p2], semis[p2])
        pltpu.async_copy(dst_hbm.at[pl.ds(off, _K)], idxc[p3], semic[p3])

    def wait_idx(c, p2, p3):
        off = chunk_off(c)
        pltpu.make_async_copy(
            src_hbm.at[pl.ds(off, _K)], idxs[p2], semis[p2]).wait()
        pltpu.make_async_copy(
            dst_hbm.at[pl.ds(off, _K)], idxc[p3], semic[p3]).wait()

    def issue_gathers(p2, p3):
        pltpu.async_copy(sctab_s.at[idxs[p2]], scs[p2], semgs[p2])
        pltpu.async_copy(sctab_s.at[idxc[p3]], scd[p2], semgd[p2])
        pltpu.async_copy(feat_hbm.at[idxs[p2]], rows[p3], semgf[p2])

    def compute(p2, p3):
        pltpu.make_async_copy(
            sctab_s.at[idxs[p2]], scs[p2], semgs[p2]).wait()
        pltpu.make_async_copy(
            sctab_s.at[idxc[p3]], scd[p2], semgd[p2]).wait()

        def ex_body(j, carry):
            rid = j * 16 + lane
            el = plsc.load_gather(scs[p2], [rid, zeros_i])
            er = plsc.load_gather(scd[p2], [rid, ones_i])
            s = el + er
            e = jnp.where(s >= 0.0, s, 0.2 * s)
            exs[pl.ds(j * 16, 16)] = jnp.exp(e - cvv)
            return carry

        lax.fori_loop(0, _K // 16, ex_body, 0)
        pltpu.make_async_copy(
            feat_hbm.at[idxs[p2]], rows[p3], semgf[p2]).wait()

        def sc_body(j, carry):
            exv = exs[pl.ds(j * 16, 16)]
            for l in range(16):
                ei = j * 16 + l
                bv = jnp.full((16,), exv[l], _F32)
                ex0[ei, pl.ds(0, 16)] = bv
                for v in range(_D // 16):
                    rows[p3][ei, pl.ds(v * 16, 16)] = (
                        rows[p3][ei, pl.ds(v * 16, 16)] * bv)
            return carry

        lax.fori_loop(0, _K // 16, sc_body, 0)

    def body(c, p2, p3, first=False, has_next=True, has_next2=True):
        q2, q3 = (p2 + 1) % 2, (p3 + 1) % 3
        if has_next:
            wait_idx(c + 1, q2, q3)
            issue_gathers(q2, q3)
        compute(p2, p3)
        pltpu.sync_copy(rows[p3], numer_s.at[idxc[p3]], add=True)
        pltpu.sync_copy(ex0, denom_s.at[idxc[p3]], add=True)
        if has_next2:
            issue_idx(c + 2, p2, (p3 + 2) % 3)

    issue_idx(0, 0, 0)
    issue_idx(1, 1, 1)
    wait_idx(0, 0, 0)
    issue_gathers(0, 0)
    body(0, 0, 0, first=True)

    def step(i, carry):
        c = 1 + 6 * i
        for j in range(6):
            body(c + j, (1 + j) % 2, (1 + j) % 3)
        return carry

    lax.fori_loop(0, (_NCH - 5) // 6, step, 0)

    body(121, 1, 1)
    body(122, 0, 2)
    body(123, 1, 0, has_next2=False)
    body(124, 0, 1, has_next=False, has_next2=False)
    plsc.subcore_barrier()

    for k in range((_NZC + _NS - 1) // _NS):
        ci = sid + _NS * k

        @pl.when(ci < _NZC)
        def _copy_chunk():
            r = pl.multiple_of(ci * _ZR, 8)
            pltpu.sync_copy(numer_s.at[pl.ds(r, _ZR)], rows0)
            pltpu.sync_copy(rows0, numer_hbm.at[cid].at[pl.ds(r, _ZR)])
            pltpu.sync_copy(denom_s.at[pl.ds(r, _ZR)], ex0)
            pltpu.sync_copy(ex0, denom_hbm.at[cid].at[pl.ds(r, _ZR)])


_edge_call = pl.kernel(
    _sc_edge_body,
    out_type=[
        jax.ShapeDtypeStruct((_NC, _N, _D), _F32),
        jax.ShapeDtypeStruct((_NC, _N, 16), _F32),
    ],
    mesh=plsc.VectorSubcoreMesh(core_axis_name="c", subcore_axis_name="s"),
    compiler_params=pltpu.CompilerParams(needs_layout_passes=False,
                                         use_tc_tiling_on_sc=False),
    scratch_types=[
        pltpu.VMEM_SHARED((_N, _D), _F32),
        pltpu.VMEM_SHARED((_N, 16), _F32),
        pltpu.VMEM_SHARED((_N, 8), _F32),
        pltpu.VMEM((_K, _D), _F32),
        pltpu.VMEM((_K, _D), _F32),
        pltpu.VMEM((_K, _D), _F32),
        pltpu.VMEM((_K, 16), _F32),
        pltpu.VMEM((_K,), jnp.int32),
        pltpu.VMEM((_K,), jnp.int32),
        pltpu.VMEM((_K,), jnp.int32),
        pltpu.VMEM((_K,), jnp.int32),
        pltpu.VMEM((_K,), jnp.int32),
        pltpu.VMEM((_K, 8), _F32),
        pltpu.VMEM((_K, 8), _F32),
        pltpu.VMEM((_K, 8), _F32),
        pltpu.VMEM((_K, 8), _F32),
        pltpu.VMEM((_K,), _F32),
        pltpu.VMEM((16,), _F32),
        pltpu.SemaphoreType.DMA,
        pltpu.SemaphoreType.DMA,
        pltpu.SemaphoreType.DMA,
        pltpu.SemaphoreType.DMA,
        pltpu.SemaphoreType.DMA,
        pltpu.SemaphoreType.DMA,
        pltpu.SemaphoreType.DMA,
        pltpu.SemaphoreType.DMA,
        pltpu.SemaphoreType.DMA,
        pltpu.SemaphoreType.DMA,
        pltpu.SemaphoreType.DMA,
    ],
)


def _post_body(numer_ref, denom_ref, h_ref, b_ref, g_ref, be_ref, out_ref,
               *, act):
    numer = numer_ref[0] + numer_ref[1]
    den = denom_ref[0, :, 0:1] + denom_ref[1, :, 0:1]
    v = numer / (den + 1e-30) + h_ref[...] + b_ref[...]
    if act:
        v = jnp.maximum(v, 0.0)
    mu = jnp.mean(v, axis=0, keepdims=True)
    var = jnp.mean((v - mu) ** 2, axis=0, keepdims=True)
    out_ref[...] = (v - mu) * lax.rsqrt(var + 1e-5) * g_ref[...] + be_ref[...]


def _make_post(act):
    return pl.pallas_call(
        functools.partial(_post_body, act=act),
        out_shape=jax.ShapeDtypeStruct((_N, _D), _F32),
    )


_post_act = _make_post(True)
_post_noact = _make_post(False)


def kernel(x, edge_index, W0, al0, ar0, b0, g0, be0,
           W1, al1, ar1, b1, g1, be1, W2, al2, ar2, b2, g2, be2):
    src = edge_index[0].astype(jnp.int32)
    dst = edge_index[1].astype(jnp.int32)
    h = x
    layers = [
        (W0, al0, ar0, b0, g0, be0, True),
        (W1, al1, ar1, b1, g1, be1, True),
        (W2, al2, ar2, b2, g2, be2, False),
    ]
    for W, al, ar, b, g, be, act in layers:
        feat, sctab, cvec = _pre_call(h, W, al.reshape(_D, 1),
                                      ar.reshape(_D, 1))
        numer, denom = _edge_call(feat, sctab, cvec, src, dst)
        post = _post_act if act else _post_noact
        h = post(numer, denom, h, b.reshape(1, _D), g.reshape(1, _D),
                 be.reshape(1, _D))
    return h


# speedup vs baseline: 1.8209x; 1.1709x over previous
import functools

import jax
import jax.numpy as jnp
from jax import lax
from jax.experimental import pallas as pl
from jax.experimental.pallas import tpu as pltpu
from jax.experimental.pallas import tpu_sc as plsc

_N = 10000
_E = 320000
_D = 128
_NC = 2
_NS = 16
_NW = _NC * _NS
_K = 80
_EPW = _E // _NW
_NCH = _EPW // _K
_ZR = 80
_NZC = _N // _ZR
_F32 = jnp.float32


def _pre_body(h_ref, w_ref, al_ref, ar_ref, feat_ref, sctab_ref, cvec_ref):
    h = h_ref[...]
    feat = jnp.dot(h, w_ref[...], preferred_element_type=_F32)
    feat_ref[...] = feat
    el = jnp.dot(feat, al_ref[...], preferred_element_type=_F32)
    er = jnp.dot(feat, ar_ref[...], preferred_element_type=_F32)
    sctab_ref[...] = jnp.concatenate(
        [el, er, jnp.zeros((_N, 6), _F32)], axis=1)
    m = jnp.max(el) + jnp.max(er)
    c = jnp.where(m >= 0.0, m, 0.2 * m)
    cvec_ref[...] = jnp.full((1, 16), c, _F32)


_pre_call = pl.pallas_call(
    _pre_body,
    out_shape=[
        jax.ShapeDtypeStruct((_N, _D), _F32),
        jax.ShapeDtypeStruct((_N, 8), _F32),
        jax.ShapeDtypeStruct((1, 16), _F32),
    ],
)


def _sc_edge_body(feat_hbm, sctab_hbm, cvec_hbm, src_hbm, dst_hbm,
                  numer_hbm, denom_hbm,
                  numer_s, denom_s, sctab_s,
                  rows0, rows1, rows2, ex0,
                  ids0, ids1, idc0, idc1, idc2,
                  scs0, scs1, scd0, scd1, exs, cv,
                  semis0, semis1, semic0, semic1, semic2,
                  semgs0, semgs1, semgd0, semgd1, semgf0, semgf1,
                  semcn0, semcn1, semcn2):
    rows = [rows0, rows1, rows2]
    idxs = [ids0, ids1]
    idxc = [idc0, idc1, idc2]
    scs = [scs0, scs1]
    scd = [scd0, scd1]
    semis = [semis0, semis1]
    semic = [semic0, semic1, semic2]
    semgs = [semgs0, semgs1]
    semgd = [semgd0, semgd1]
    semgf = [semgf0, semgf1]
    semcn = [semcn0, semcn1, semcn2]

    cid = lax.axis_index("c")
    sid = lax.axis_index("s")
    wid = cid * _NS + sid
    zero16 = jnp.zeros((16,), _F32)
    lane = lax.iota(jnp.int32, 16)
    lane_hi = lane // 8
    lane_lo = lane % 8

    def zb_body(i, carry):
        for v in range(_D // 16):
            rows0[i, pl.ds(v * 16, 16)] = zero16
        return carry

    lax.fori_loop(0, _ZR, zb_body, 0)

    def zs_body(i, carry):
        ex0[i, pl.ds(0, 16)] = zero16
        return carry

    lax.fori_loop(0, _K, zs_body, 0)
    for k in range((_NZC + _NS - 1) // _NS):
        ci = sid + _NS * k

        @pl.when(ci < _NZC)
        def _zero_chunk():
            r = pl.multiple_of(ci * _ZR, 8)
            pltpu.sync_copy(rows0, numer_s.at[pl.ds(r, _ZR)])
            pltpu.sync_copy(ex0, denom_s.at[pl.ds(r, _ZR)])
            pltpu.sync_copy(sctab_hbm.at[pl.ds(r, _ZR)], scs0)
            pltpu.sync_copy(scs0, sctab_s.at[pl.ds(r, _ZR)])

    pltpu.sync_copy(cvec_hbm.at[0], cv)
    plsc.subcore_barrier()

    cvv = cv[pl.ds(0, 16)]
    base = wid * _EPW
    zeros_i = jnp.zeros((16,), jnp.int32)
    ones_i = jnp.ones((16,), jnp.int32)

    def chunk_off(c):
        return pl.multiple_of(base + c * _K, 8)

    def issue_idx(c, p2, p3):
        off = chunk_off(c)
        pltpu.async_copy(src_hbm.at[pl.ds(off, _K)], idxs[p2], semis[p2])
        pltpu.async_copy(dst_hbm.at[pl.ds(off, _K)], idxc[p3], semic[p3])

    def wait_idx(c, p2, p3):
        off = chunk_off(c)
        pltpu.make_async_copy(
            src_hbm.at[pl.ds(off, _K)], idxs[p2], semis[p2]).wait()
        pltpu.make_async_copy(
            dst_hbm.at[pl.ds(off, _K)], idxc[p3], semic[p3]).wait()

    def issue_gathers(p2, p3):
        pltpu.async_copy(sctab_s.at[idxs[p2]], scs[p2], semgs[p2])
        pltpu.async_copy(sctab_s.at[idxc[p3]], scd[p2], semgd[p2])
        pltpu.async_copy(feat_hbm.at[idxs[p2]], rows[p3], semgf[p2])

    def compute(p2, p3):
        pltpu.make_async_copy(
            sctab_s.at[idxs[p2]], scs[p2], semgs[p2]).wait()
        pltpu.make_async_copy(
            sctab_s.at[idxc[p3]], scd[p2], semgd[p2]).wait()

        def ex_body(j, carry):
            rid = j * 16 + lane
            el = plsc.load_gather(scs[p2], [rid, zeros_i])
            er = plsc.load_gather(scd[p2], [rid, ones_i])
            s = el + er
            e = jnp.where(s >= 0.0, s, 0.2 * s)
            exs[pl.ds(j * 16, 16)] = jnp.exp(e - cvv)
            return carry

        lax.fori_loop(0, _K // 16, ex_body, 0)
        pltpu.make_async_copy(
            feat_hbm.at[idxs[p2]], rows[p3], semgf[p2]).wait()

        def sc_body(j, carry):
            exv = exs[pl.ds(j * 16, 16)]
            for l in range(16):
                ei = j * 16 + l
                bv = jnp.full((16,), exv[l], _F32)
                ex0[ei, pl.ds(0, 16)] = bv
                for v in range(_D // 16):
                    rows[p3][ei, pl.ds(v * 16, 16)] = (
                        rows[p3][ei, pl.ds(v * 16, 16)] * bv)
            return carry

        lax.fori_loop(0, _K // 16, sc_body, 0)

    def body(c, p2, p3, first=False, has_next=True, has_next2=True):
        q2, q3 = (p2 + 1) % 2, (p3 + 1) % 3
        if has_next:
            wait_idx(c + 1, q2, q3)
            issue_gathers(q2, q3)
        compute(p2, p3)
        if not first:
            pm = (p3 + 2) % 3
            pltpu.make_async_copy(
                rows[pm], numer_s.at[idxc[pm]], semcn[pm]).wait()
        pltpu.async_copy(rows[p3], numer_s.at[idxc[p3]], semcn[p3],
                         add=True)
        pltpu.sync_copy(ex0, denom_s.at[idxc[p3]], add=True)
        if has_next2:
            issue_idx(c + 2, p2, (p3 + 2) % 3)

    issue_idx(0, 0, 0)
    issue_idx(1, 1, 1)
    wait_idx(0, 0, 0)
    issue_gathers(0, 0)
    body(0, 0, 0, first=True)

    def step(i, carry):
        c = 1 + 6 * i
        for j in range(6):
            body(c + j, (1 + j) % 2, (1 + j) % 3)
        return carry

    lax.fori_loop(0, (_NCH - 5) // 6, step, 0)

    body(121, 1, 1)
    body(122, 0, 2)
    body(123, 1, 0, has_next2=False)
    body(124, 0, 1, has_next=False, has_next2=False)
    pltpu.make_async_copy(rows[1], numer_s.at[idxc[1]], semcn[1]).wait()
    plsc.subcore_barrier()

    for k in range((_NZC + _NS - 1) // _NS):
        ci = sid + _NS * k

        @pl.when(ci < _NZC)
        def _copy_chunk():
            r = pl.multiple_of(ci * _ZR, 8)
            pltpu.sync_copy(numer_s.at[pl.ds(r, _ZR)], rows0)
            pltpu.sync_copy(rows0, numer_hbm.at[cid].at[pl.ds(r, _ZR)])
            pltpu.sync_copy(denom_s.at[pl.ds(r, _ZR)], ex0)
            pltpu.sync_copy(ex0, denom_hbm.at[cid].at[pl.ds(r, _ZR)])


_edge_call = pl.kernel(
    _sc_edge_body,
    out_type=[
        jax.ShapeDtypeStruct((_NC, _N, _D), _F32),
        jax.ShapeDtypeStruct((_NC, _N, 16), _F32),
    ],
    mesh=plsc.VectorSubcoreMesh(core_axis_name="c", subcore_axis_name="s"),
    compiler_params=pltpu.CompilerParams(needs_layout_passes=False,
                                         use_tc_tiling_on_sc=False),
    scratch_types=[
        pltpu.VMEM_SHARED((_N, _D), _F32),
        pltpu.VMEM_SHARED((_N, 16), _F32),
        pltpu.VMEM_SHARED((_N, 8), _F32),
        pltpu.VMEM((_K, _D), _F32),
        pltpu.VMEM((_K, _D), _F32),
        pltpu.VMEM((_K, _D), _F32),
        pltpu.VMEM((_K, 16), _F32),
        pltpu.VMEM((_K,), jnp.int32),
        pltpu.VMEM((_K,), jnp.int32),
        pltpu.VMEM((_K,), jnp.int32),
        pltpu.VMEM((_K,), jnp.int32),
        pltpu.VMEM((_K,), jnp.int32),
        pltpu.VMEM((_K, 8), _F32),
        pltpu.VMEM((_K, 8), _F32),
        pltpu.VMEM((_K, 8), _F32),
        pltpu.VMEM((_K, 8), _F32),
        pltpu.VMEM((_K,), _F32),
        pltpu.VMEM((16,), _F32),
        pltpu.SemaphoreType.DMA,
        pltpu.SemaphoreType.DMA,
        pltpu.SemaphoreType.DMA,
        pltpu.SemaphoreType.DMA,
        pltpu.SemaphoreType.DMA,
        pltpu.SemaphoreType.DMA,
        pltpu.SemaphoreType.DMA,
        pltpu.SemaphoreType.DMA,
        pltpu.SemaphoreType.DMA,
        pltpu.SemaphoreType.DMA,
        pltpu.SemaphoreType.DMA,
        pltpu.SemaphoreType.DMA,
        pltpu.SemaphoreType.DMA,
        pltpu.SemaphoreType.DMA,
    ],
)


def _post_body(numer_ref, denom_ref, h_ref, b_ref, g_ref, be_ref, out_ref,
               *, act):
    numer = numer_ref[0] + numer_ref[1]
    den = denom_ref[0, :, 0:1] + denom_ref[1, :, 0:1]
    v = numer / (den + 1e-30) + h_ref[...] + b_ref[...]
    if act:
        v = jnp.maximum(v, 0.0)
    mu = jnp.mean(v, axis=0, keepdims=True)
    var = jnp.mean((v - mu) ** 2, axis=0, keepdims=True)
    out_ref[...] = (v - mu) * lax.rsqrt(var + 1e-5) * g_ref[...] + be_ref[...]


def _make_post(act):
    return pl.pallas_call(
        functools.partial(_post_body, act=act),
        out_shape=jax.ShapeDtypeStruct((_N, _D), _F32),
    )


_post_act = _make_post(True)
_post_noact = _make_post(False)


def kernel(x, edge_index, W0, al0, ar0, b0, g0, be0,
           W1, al1, ar1, b1, g1, be1, W2, al2, ar2, b2, g2, be2):
    src = edge_index[0].astype(jnp.int32)
    dst = edge_index[1].astype(jnp.int32)
    h = x
    layers = [
        (W0, al0, ar0, b0, g0, be0, True),
        (W1, al1, ar1, b1, g1, be1, True),
        (W2, al2, ar2, b2, g2, be2, False),
    ]
    for W, al, ar, b, g, be, act in layers:
        feat, sctab, cvec = _pre_call(h, W, al.reshape(_D, 1),
                                      ar.reshape(_D, 1))
        numer, denom = _edge_call(feat, sctab, cvec, src, dst)
        post = _post_act if act else _post_noact
        h = post(numer, denom, h, b.reshape(1, _D), g.reshape(1, _D),
                 be.reshape(1, _D))
    return h


# speedup vs baseline: 1.9465x; 1.0690x over previous
import functools

import jax
import jax.numpy as jnp
from jax import lax
from jax.experimental import pallas as pl
from jax.experimental.pallas import tpu as pltpu
from jax.experimental.pallas import tpu_sc as plsc

_N = 10000
_E = 320000
_D = 128
_NC = 2
_NS = 16
_NW = _NC * _NS
_K = 80
_EPW = _E // _NW
_NCH = _EPW // _K
_ZR = 80
_NZC = _N // _ZR
_F32 = jnp.float32


def _pre_body(h_ref, w_ref, al_ref, ar_ref, feat_ref, sctab_ref, cvec_ref):
    h = h_ref[...]
    feat = jnp.dot(h, w_ref[...], preferred_element_type=_F32)
    feat_ref[...] = feat
    el = jnp.dot(feat, al_ref[...], preferred_element_type=_F32)
    er = jnp.dot(feat, ar_ref[...], preferred_element_type=_F32)
    sctab_ref[...] = jnp.concatenate(
        [el, er, jnp.zeros((_N, 6), _F32)], axis=1)
    m = jnp.max(el) + jnp.max(er)
    c = jnp.where(m >= 0.0, m, 0.2 * m)
    cvec_ref[...] = jnp.full((1, 16), c, _F32)


_pre_call = pl.pallas_call(
    _pre_body,
    out_shape=[
        jax.ShapeDtypeStruct((_N, _D), _F32),
        jax.ShapeDtypeStruct((_N, 8), _F32),
        jax.ShapeDtypeStruct((1, 16), _F32),
    ],
)


def _sc_edge_body(feat_hbm, sctab_hbm, cvec_hbm, src_hbm, dst_hbm,
                  numer_hbm, denom_hbm,
                  numer_s, denom_s, sctab_s,
                  rows0, rows1, rows2, ex0,
                  ids0, ids1, idc0, idc1, idc2,
                  scs0, scs1, scd0, scd1, exs, cv,
                  semis0, semis1, semic0, semic1, semic2,
                  semgs0, semgs1, semgd0, semgd1, semgf0, semgf1,
                  semcn0, semcn1, semcn2, semcd0):
    rows = [rows0, rows1, rows2]
    idxs = [ids0, ids1]
    idxc = [idc0, idc1, idc2]
    scs = [scs0, scs1]
    scd = [scd0, scd1]
    semis = [semis0, semis1]
    semic = [semic0, semic1, semic2]
    semgs = [semgs0, semgs1]
    semgd = [semgd0, semgd1]
    semgf = [semgf0, semgf1]
    semcn = [semcn0, semcn1, semcn2]
    semcd = [semcd0]

    cid = lax.axis_index("c")
    sid = lax.axis_index("s")
    wid = cid * _NS + sid
    zero16 = jnp.zeros((16,), _F32)
    lane = lax.iota(jnp.int32, 16)
    lane_hi = lane // 8
    lane_lo = lane % 8

    def zb_body(i, carry):
        for v in range(_D // 16):
            rows0[i, pl.ds(v * 16, 16)] = zero16
        return carry

    lax.fori_loop(0, _ZR, zb_body, 0)

    def zs_body(i, carry):
        ex0[i, pl.ds(0, 16)] = zero16
        return carry

    lax.fori_loop(0, _K, zs_body, 0)
    for k in range((_NZC + _NS - 1) // _NS):
        ci = sid + _NS * k

        @pl.when(ci < _NZC)
        def _zero_chunk():
            r = pl.multiple_of(ci * _ZR, 8)
            pltpu.sync_copy(rows0, numer_s.at[pl.ds(r, _ZR)])
            pltpu.sync_copy(ex0, denom_s.at[pl.ds(r, _ZR)])
            pltpu.sync_copy(sctab_hbm.at[pl.ds(r, _ZR)], scs0)
            pltpu.sync_copy(scs0, sctab_s.at[pl.ds(r, _ZR)])

    pltpu.sync_copy(cvec_hbm.at[0], cv)
    plsc.subcore_barrier()

    cvv = cv[pl.ds(0, 16)]
    base = wid * _EPW
    zeros_i = jnp.zeros((16,), jnp.int32)
    ones_i = jnp.ones((16,), jnp.int32)

    def chunk_off(c):
        return pl.multiple_of(base + c * _K, 8)

    def issue_idx(c, p2, p3):
        off = chunk_off(c)
        pltpu.async_copy(src_hbm.at[pl.ds(off, _K)], idxs[p2], semis[p2])
        pltpu.async_copy(dst_hbm.at[pl.ds(off, _K)], idxc[p3], semic[p3])

    def wait_idx(c, p2, p3):
        off = chunk_off(c)
        pltpu.make_async_copy(
            src_hbm.at[pl.ds(off, _K)], idxs[p2], semis[p2]).wait()
        pltpu.make_async_copy(
            dst_hbm.at[pl.ds(off, _K)], idxc[p3], semic[p3]).wait()

    def issue_gathers(p2, p3):
        pltpu.async_copy(sctab_s.at[idxs[p2]], scs[p2], semgs[p2])
        pltpu.async_copy(sctab_s.at[idxc[p3]], scd[p2], semgd[p2])
        pltpu.async_copy(feat_hbm.at[idxs[p2]], rows[p3], semgf[p2])

    def compute(p2, p3):
        pltpu.make_async_copy(
            sctab_s.at[idxs[p2]], scs[p2], semgs[p2]).wait()
        pltpu.make_async_copy(
            sctab_s.at[idxc[p3]], scd[p2], semgd[p2]).wait()

        def ex_body(j, carry):
            rid = j * 16 + lane
            el = plsc.load_gather(scs[p2], [rid, zeros_i])
            er = plsc.load_gather(scd[p2], [rid, ones_i])
            s = el + er
            e = jnp.where(s >= 0.0, s, 0.2 * s)
            exs[pl.ds(j * 16, 16)] = jnp.exp(e - cvv)
            return carry

        lax.fori_loop(0, _K // 16, ex_body, 0)
        pltpu.make_async_copy(
            feat_hbm.at[idxs[p2]], rows[p3], semgf[p2]).wait()

        def sc_body(j, carry):
            exv = exs[pl.ds(j * 16, 16)]
            for l in range(16):
                ei = j * 16 + l
                bv = jnp.full((16,), exv[l], _F32)
                ex0[ei, pl.ds(0, 16)] = bv
                for v in range(_D // 16):
                    rows[p3][ei, pl.ds(v * 16, 16)] = (
                        rows[p3][ei, pl.ds(v * 16, 16)] * bv)
            return carry

        lax.fori_loop(0, _K // 16, sc_body, 0)

    def body(c, p2, p3, first=False, has_next=True, has_next2=True):
        q2, q3 = (p2 + 1) % 2, (p3 + 1) % 3
        pm = (p3 + 2) % 3
        if has_next:
            wait_idx(c + 1, q2, q3)
            issue_gathers(q2, q3)
        if not first:
            pltpu.make_async_copy(
                ex0, denom_s.at[idxc[pm]], semcd[0]).wait()
        compute(p2, p3)
        if not first:
            pltpu.make_async_copy(
                rows[pm], numer_s.at[idxc[pm]], semcn[pm]).wait()
        pltpu.async_copy(rows[p3], numer_s.at[idxc[p3]], semcn[p3],
                         add=True)
        pltpu.async_copy(ex0, denom_s.at[idxc[p3]], semcd[0], add=True)
        if has_next2:
            issue_idx(c + 2, p2, (p3 + 2) % 3)

    issue_idx(0, 0, 0)
    issue_idx(1, 1, 1)
    wait_idx(0, 0, 0)
    issue_gathers(0, 0)
    body(0, 0, 0, first=True)

    def step(i, carry):
        c = 1 + 6 * i
        for j in range(6):
            body(c + j, (1 + j) % 2, (1 + j) % 3)
        return carry

    lax.fori_loop(0, (_NCH - 5) // 6, step, 0)

    body(121, 1, 1)
    body(122, 0, 2)
    body(123, 1, 0, has_next2=False)
    body(124, 0, 1, has_next=False, has_next2=False)
    pltpu.make_async_copy(rows[1], numer_s.at[idxc[1]], semcn[1]).wait()
    pltpu.make_async_copy(ex0, denom_s.at[idxc[1]], semcd[0]).wait()
    plsc.subcore_barrier()

    for k in range((_NZC + _NS - 1) // _NS):
        ci = sid + _NS * k

        @pl.when(ci < _NZC)
        def _copy_chunk():
            r = pl.multiple_of(ci * _ZR, 8)
            pltpu.sync_copy(numer_s.at[pl.ds(r, _ZR)], rows0)
            pltpu.sync_copy(rows0, numer_hbm.at[cid].at[pl.ds(r, _ZR)])
            pltpu.sync_copy(denom_s.at[pl.ds(r, _ZR)], ex0)
            pltpu.sync_copy(ex0, denom_hbm.at[cid].at[pl.ds(r, _ZR)])


_edge_call = pl.kernel(
    _sc_edge_body,
    out_type=[
        jax.ShapeDtypeStruct((_NC, _N, _D), _F32),
        jax.ShapeDtypeStruct((_NC, _N, 16), _F32),
    ],
    mesh=plsc.VectorSubcoreMesh(core_axis_name="c", subcore_axis_name="s"),
    compiler_params=pltpu.CompilerParams(needs_layout_passes=False,
                                         use_tc_tiling_on_sc=False),
    scratch_types=[
        pltpu.VMEM_SHARED((_N, _D), _F32),
        pltpu.VMEM_SHARED((_N, 16), _F32),
        pltpu.VMEM_SHARED((_N, 8), _F32),
        pltpu.VMEM((_K, _D), _F32),
        pltpu.VMEM((_K, _D), _F32),
        pltpu.VMEM((_K, _D), _F32),
        pltpu.VMEM((_K, 16), _F32),
        pltpu.VMEM((_K,), jnp.int32),
        pltpu.VMEM((_K,), jnp.int32),
        pltpu.VMEM((_K,), jnp.int32),
        pltpu.VMEM((_K,), jnp.int32),
        pltpu.VMEM((_K,), jnp.int32),
        pltpu.VMEM((_K, 8), _F32),
        pltpu.VMEM((_K, 8), _F32),
        pltpu.VMEM((_K, 8), _F32),
        pltpu.VMEM((_K, 8), _F32),
        pltpu.VMEM((_K,), _F32),
        pltpu.VMEM((16,), _F32),
        pltpu.SemaphoreType.DMA,
        pltpu.SemaphoreType.DMA,
        pltpu.SemaphoreType.DMA,
        pltpu.SemaphoreType.DMA,
        pltpu.SemaphoreType.DMA,
        pltpu.SemaphoreType.DMA,
        pltpu.SemaphoreType.DMA,
        pltpu.SemaphoreType.DMA,
        pltpu.SemaphoreType.DMA,
        pltpu.SemaphoreType.DMA,
        pltpu.SemaphoreType.DMA,
        pltpu.SemaphoreType.DMA,
        pltpu.SemaphoreType.DMA,
        pltpu.SemaphoreType.DMA,
        pltpu.SemaphoreType.DMA,
    ],
)


def _post_body(numer_ref, denom_ref, h_ref, b_ref, g_ref, be_ref, out_ref,
               *, act):
    numer = numer_ref[0] + numer_ref[1]
    den = denom_ref[0, :, 0:1] + denom_ref[1, :, 0:1]
    v = numer / (den + 1e-30) + h_ref[...] + b_ref[...]
    if act:
        v = jnp.maximum(v, 0.0)
    mu = jnp.mean(v, axis=0, keepdims=True)
    var = jnp.mean((v - mu) ** 2, axis=0, keepdims=True)
    out_ref[...] = (v - mu) * lax.rsqrt(var + 1e-5) * g_ref[...] + be_ref[...]


def _make_post(act):
    return pl.pallas_call(
        functools.partial(_post_body, act=act),
        out_shape=jax.ShapeDtypeStruct((_N, _D), _F32),
    )


_post_act = _make_post(True)
_post_noact = _make_post(False)


def kernel(x, edge_index, W0, al0, ar0, b0, g0, be0,
           W1, al1, ar1, b1, g1, be1, W2, al2, ar2, b2, g2, be2):
    src = edge_index[0].astype(jnp.int32)
    dst = edge_index[1].astype(jnp.int32)
    h = x
    layers = [
        (W0, al0, ar0, b0, g0, be0, True),
        (W1, al1, ar1, b1, g1, be1, True),
        (W2, al2, ar2, b2, g2, be2, False),
    ]
    for W, al, ar, b, g, be, act in layers:
        feat, sctab, cvec = _pre_call(h, W, al.reshape(_D, 1),
                                      ar.reshape(_D, 1))
        numer, denom = _edge_call(feat, sctab, cvec, src, dst)
        post = _post_act if act else _post_noact
        h = post(numer, denom, h, b.reshape(1, _D), g.reshape(1, _D),
                 be.reshape(1, _D))
    return h
